# baseline (speedup 1.0000x reference)
import functools

import jax
import jax.numpy as jnp
from jax import lax
from jax.experimental import pallas as pl
from jax.experimental.pallas import tpu as pltpu
from jax.experimental.pallas import tpu_sc as plsc

_NC = 2
_NS = 16
_NW = _NC * _NS

_D = 64
_T = 8
_KT = 50
_CHUNK = 80
_WB = 400
_CPW = _WB // _CHUNK


def _make_pack(num_tiles):
    n_chunks_total = num_tiles // _KT
    per_w = -(-n_chunks_total // _NW)

    mesh = plsc.VectorSubcoreMesh(
        core_axis_name="c", subcore_axis_name="s",
        num_cores=_NC, num_subcores=_NS)

    @functools.partial(
        pl.kernel,
        mesh=mesh,
        compiler_params=pltpu.CompilerParams(needs_layout_passes=False),
        out_type=jax.ShapeDtypeStruct((num_tiles * _T // 2, 2 * _D),
                                      jnp.float32),
        scratch_types=[
            pltpu.VMEM((_KT, _T, _D), jnp.float32),
            pltpu.VMEM((_KT * _T // 2, 2 * _D), jnp.float32),
        ],
    )
    def pack(table_hbm, packed_hbm, tiles_v, pairs_v):
        wid = lax.axis_index("s") * _NC + lax.axis_index("c")
        lanes = lax.iota(jnp.int32, 16)

        def chunk_body(m, _):
            ci = wid * per_w + m

            @pl.when(ci < n_chunks_total)
            def _():
                t0 = pl.multiple_of(ci * _KT, 2)
                pltpu.sync_copy(table_hbm.at[pl.ds(t0, _KT)], tiles_v)

                def tile_body(i, _):
                    iv = jnp.broadcast_to(i, (16,)).astype(jnp.int32)
                    for s in range(_T):
                        sv = jnp.full((16,), s, jnp.int32)
                        for cg in range(_D // 16):
                            vals = plsc.load_gather(
                                tiles_v, [iv, sv, lanes + cg * 16])
                            pairs_v[i * (_T // 2) + (s // 2),
                                    pl.ds((s % 2) * _D + cg * 16, 16)] = vals
                    return 0

                lax.fori_loop(0, _KT, tile_body, 0)
                u0 = pl.multiple_of(ci * (_KT * _T // 2), 8)
                pltpu.sync_copy(pairs_v,
                                packed_hbm.at[pl.ds(u0, _KT * _T // 2)])
            return 0

        lax.fori_loop(0, per_w, chunk_body, 0)

    return pack


def _make_gather(total):
    n_per_w = total // _NW
    n_chunks = n_per_w // _CHUNK

    mesh = plsc.VectorSubcoreMesh(
        core_axis_name="c", subcore_axis_name="s",
        num_cores=_NC, num_subcores=_NS)

    @functools.partial(
        pl.kernel,
        mesh=mesh,
        compiler_params=pltpu.CompilerParams(needs_layout_passes=False),
        out_type=jax.ShapeDtypeStruct((total * _D,), jnp.float32),
        scratch_types=[
            pltpu.VMEM((n_per_w,), jnp.int32),
            pltpu.VMEM((_CHUNK,), jnp.int32),
            pltpu.VMEM((_CHUNK,), jnp.int32),
            pltpu.VMEM((_CHUNK, 2 * _D), jnp.float32),
            pltpu.VMEM((_CHUNK, 2 * _D), jnp.float32),
            pltpu.VMEM((_WB * _D,), jnp.float32),
            pltpu.SemaphoreType.DMA,
            pltpu.SemaphoreType.DMA,
        ],
    )
    def gather(table_hbm, idx_hbm, out_hbm, idx_v, pid_a, pid_b,
               pairs_a, pairs_b, rows_v, sem_a, sem_b):
        wid = lax.axis_index("s") * _NC + lax.axis_index("c")
        base = pl.multiple_of(wid * n_per_w, 128)
        pltpu.sync_copy(idx_hbm.at[pl.ds(base, n_per_w)], idx_v)

        pids = (pid_a, pid_b)
        pairs = (pairs_a, pairs_b)
        sems = (sem_a, sem_b)
        lanes = lax.iota(jnp.int32, 16)

        def issue(j, slot):
            off = j * _CHUNK
            for i in range(_CHUNK // 16):
                v = idx_v[pl.ds(off + i * 16, 16)]
                pids[slot][pl.ds(i * 16, 16)] = lax.shift_right_logical(v, 1)
            pltpu.async_copy(table_hbm.at[pids[slot]], pairs[slot],
                             sems[slot])

        def extract(j, slot):
            off = j * _CHUNK
            gbase = lax.rem(lax.mul(j, _CHUNK), _WB)
            for i in range(_CHUNK // 16):
                rvec = lanes + i * 16
                ivec = idx_v[pl.ds(off + i * 16, 16)]
                halfc = lax.mul(lax.bitwise_and(ivec, 1), _D)
                dst0 = lax.mul(gbase + i * 16, _D) + lanes * _D
                for col in range(_D):
                    vals = plsc.load_gather(pairs[slot], [rvec, halfc + col])
                    plsc.store_scatter(rows_v, [dst0 + col], vals)

        def flush(j):
            wb_i = j // _CPW
            r0 = pl.multiple_of((base + wb_i * _WB) * _D, 128)
            pltpu.sync_copy(rows_v, out_hbm.at[pl.ds(r0, _WB * _D)])

        issue(0, 0)

        def pair_body(m, carry):
            j0 = m * 2
            issue(j0 + 1, 1)
            pltpu.make_async_copy(table_hbm.at[pids[0]], pairs[0],
                                  sems[0]).wait()
            extract(j0, 0)

            @pl.when(lax.rem(j0, _CPW) == _CPW - 1)
            def _():
                flush(j0)

            @pl.when(j0 + 2 < n_chunks)
            def _():
                issue(j0 + 2, 0)

            pltpu.make_async_copy(table_hbm.at[pids[1]], pairs[1],
                                  sems[1]).wait()
            extract(j0 + 1, 1)

            @pl.when(lax.rem(j0 + 1, _CPW) == _CPW - 1)
            def _():
                flush(j0 + 1)
            return 0

        lax.fori_loop(0, n_chunks // 2, pair_body, 0)

    return gather


def kernel(x, index, latents):
    b, h = index.shape
    num_rows, d = latents.shape
    table3d = latents.reshape(num_rows // _T, _T, d)
    packed = _make_pack(num_rows // _T)(table3d)
    idx_flat = index.reshape(b * h).astype(jnp.int32)
    flat = _make_gather(b * h)(packed, idx_flat)
    return flat.reshape(b, h, d)


# speedup vs baseline: 2.1325x; 2.1325x over previous
import functools

import jax
import jax.numpy as jnp
from jax import lax
from jax.experimental import pallas as pl
from jax.experimental.pallas import tpu as pltpu
from jax.experimental.pallas import tpu_sc as plsc

_NC = 2
_NS = 16
_NW = _NC * _NS

_D = 64
_CHUNK = 800


def _make_gather(total, dtype):
    n_per_w = total // _NW
    n_chunks = n_per_w // _CHUNK
    assert n_chunks * _CHUNK == n_per_w

    mesh = plsc.VectorSubcoreMesh(
        core_axis_name="c", subcore_axis_name="s",
        num_cores=_NC, num_subcores=_NS)

    @functools.partial(
        pl.kernel,
        mesh=mesh,
        compiler_params=pltpu.CompilerParams(use_tc_tiling_on_sc=False),
        out_type=jax.ShapeDtypeStruct((total, _D), dtype),
        scratch_types=[
            pltpu.VMEM((n_per_w,), jnp.int32),
            pltpu.VMEM((_CHUNK, _D), dtype),
            pltpu.VMEM((_CHUNK, _D), dtype),
            pltpu.SemaphoreType.DMA,
            pltpu.SemaphoreType.DMA,
        ],
    )
    def gather(table_hbm, idx_hbm, out_hbm, idx_v, rows_a, rows_b, sem_a, sem_b):
        wid = lax.axis_index("s") * _NC + lax.axis_index("c")
        base = wid * n_per_w
        pltpu.sync_copy(idx_hbm.at[pl.ds(base, n_per_w)], idx_v)
        bufs = (rows_a, rows_b)
        sems = (sem_a, sem_b)
        pend = [None, None]
        pend[0] = pltpu.async_copy(
            table_hbm.at[idx_v.at[pl.ds(0, _CHUNK)]], bufs[0], sems[0])
        for j in range(1, n_chunks):
            pend[j % 2] = pltpu.async_copy(
                table_hbm.at[idx_v.at[pl.ds(j * _CHUNK, _CHUNK)]],
                bufs[j % 2], sems[j % 2])
            pend[(j - 1) % 2].wait()
            pltpu.sync_copy(bufs[(j - 1) % 2],
                            out_hbm.at[pl.ds(base + (j - 1) * _CHUNK, _CHUNK)])
        last = n_chunks - 1
        pend[last % 2].wait()
        pltpu.sync_copy(bufs[last % 2],
                        out_hbm.at[pl.ds(base + last * _CHUNK, _CHUNK)])

    return gather


def kernel(x, index, latents):
    b, h = index.shape
    num_rows, d = latents.shape
    idx_flat = index.reshape(b * h).astype(jnp.int32)
    out = _make_gather(b * h, latents.dtype)(latents, idx_flat)
    return out.reshape(b, h, d)
